# Initial kernel scaffold; baseline (speedup 1.0000x reference)
import jax
import jax.numpy as jnp
from jax.experimental import pallas as pl


def kernel(h_V, h_E, edge_idx, W_msg0, b_msg0, W_d0, b_d0, W_out, b_out, ln1_w, ln1_b, ln2_w, ln2_b):
    raise NotImplementedError("write your pallas kernel here")


# speedup vs baseline: 2.0439x; 2.0439x over previous
import functools

import jax
import jax.numpy as jnp
from jax import lax
from jax.experimental import pallas as pl
from jax.experimental.pallas import tpu as pltpu
from jax.experimental.pallas import tpu_sc as plsc

_N, _E, _H, _HN = 10000, 320000, 128, 144
_BE = 2000
_BN = 2000
_NC, _NS = 2, 16
_EPW = _E // (_NC * _NS)
_C = 80
_NCH = _EPW // _C
_RPS = 624
_RLAST = _N - 15 * _RPS


def _gelu(x):
    return x * 0.5 * (1.0 + lax.erf(x * 0.7071067811865476))


def _edge_mlp_body(he_ref, w_ref, b_ref, out_ref):
    x = jnp.dot(he_ref[...], w_ref[...], preferred_element_type=jnp.float32)
    out_ref[...] = _gelu(x + b_ref[...])


def _edge_mlp(h_E, W, b):
    return pl.pallas_call(
        _edge_mlp_body,
        grid=(_E // _BE,),
        in_specs=[
            pl.BlockSpec((_BE, _HN), lambda i: (i, 0)),
            pl.BlockSpec((_HN, _H), lambda i: (0, 0)),
            pl.BlockSpec((1, _H), lambda i: (0, 0)),
        ],
        out_specs=pl.BlockSpec((_BE, _H), lambda i: (i, 0)),
        out_shape=jax.ShapeDtypeStruct((_E, _H), jnp.float32),
    )(h_E, W, b.reshape(1, _H))


def _scatter_body(msg_hbm, idx_hbm, zero_hbm, out_hbm, rows_v, idx_v, acc_sh):
    c = lax.axis_index("c")
    s = lax.axis_index("s")
    r0 = pl.multiple_of(s * _RPS, 8)

    @pl.when(s < _NS - 1)
    def _():
        pltpu.sync_copy(zero_hbm.at[pl.ds(r0, _RPS)],
                        acc_sh.at[pl.ds(r0, _RPS)])

    @pl.when(s == _NS - 1)
    def _():
        pltpu.sync_copy(zero_hbm.at[pl.ds(r0, _RLAST)],
                        acc_sh.at[pl.ds(r0, _RLAST)])

    plsc.subcore_barrier()
    base = (s * _NC + c) * _EPW

    def body(i, carry):
        off = base + i * _C
        pltpu.sync_copy(idx_hbm.at[pl.ds(off, _C)], idx_v)
        pltpu.sync_copy(msg_hbm.at[pl.ds(off, _C)], rows_v)
        pltpu.sync_copy(rows_v, acc_sh.at[idx_v], add=True)
        return carry

    lax.fori_loop(0, _NCH, body, 0)
    plsc.subcore_barrier()

    @pl.when(s < _NS - 1)
    def _():
        pltpu.sync_copy(acc_sh.at[pl.ds(r0, _RPS)],
                        out_hbm.at[c, pl.ds(r0, _RPS)])

    @pl.when(s == _NS - 1)
    def _():
        pltpu.sync_copy(acc_sh.at[pl.ds(r0, _RLAST)],
                        out_hbm.at[c, pl.ds(r0, _RLAST)])


def _scatter(msg, src_idx, zeros):
    f = pl.kernel(
        _scatter_body,
        out_type=jax.ShapeDtypeStruct((_NC, _N, _H), jnp.float32),
        mesh=plsc.VectorSubcoreMesh(core_axis_name="c", subcore_axis_name="s"),
        scratch_types=[
            pltpu.VMEM((_C, _H), jnp.float32),
            pltpu.VMEM((_C,), jnp.int32),
            pltpu.VMEM_SHARED((_N, _H), jnp.float32),
        ],
    )
    return f(msg, src_idx, zeros)


def _final_body(hv_ref, p0_ref, p1_ref, wd_ref, bd_ref, wo_ref, bo_ref,
                ln1w_ref, ln1b_ref, ln2w_ref, ln2b_ref, out_ref):
    x = hv_ref[...] + (p0_ref[...] + p1_ref[...]) * (1.0 / 30.0)
    mu = jnp.mean(x, axis=-1, keepdims=True)
    xc = x - mu
    var = jnp.mean(xc * xc, axis=-1, keepdims=True)
    xn = xc * lax.rsqrt(var + 1e-5) * ln1w_ref[...] + ln1b_ref[...]
    h = _gelu(jnp.dot(xn, wd_ref[...], preferred_element_type=jnp.float32)
              + bd_ref[...])
    y = xn + jnp.dot(h, wo_ref[...], preferred_element_type=jnp.float32) \
        + bo_ref[...]
    mu2 = jnp.mean(y, axis=-1, keepdims=True)
    yc = y - mu2
    var2 = jnp.mean(yc * yc, axis=-1, keepdims=True)
    out_ref[...] = yc * lax.rsqrt(var2 + 1e-5) * ln2w_ref[...] + ln2b_ref[...]


def _final(h_V, p0, p1, W_d0, b_d0, W_out, b_out, ln1_w, ln1_b, ln2_w, ln2_b):
    row = pl.BlockSpec((_BN, _H), lambda i: (i, 0))
    full = pl.BlockSpec((_H, _H), lambda i: (0, 0))
    vec = pl.BlockSpec((1, _H), lambda i: (0, 0))
    return pl.pallas_call(
        _final_body,
        grid=(_N // _BN,),
        in_specs=[row, row, row, full, vec, full, vec, vec, vec, vec, vec],
        out_specs=row,
        out_shape=jax.ShapeDtypeStruct((_N, _H), jnp.float32),
    )(h_V, p0, p1, W_d0, b_d0.reshape(1, _H), W_out, b_out.reshape(1, _H),
      ln1_w.reshape(1, _H), ln1_b.reshape(1, _H),
      ln2_w.reshape(1, _H), ln2_b.reshape(1, _H))


def kernel(h_V, h_E, edge_idx, W_msg0, b_msg0, W_d0, b_d0, W_out, b_out,
           ln1_w, ln1_b, ln2_w, ln2_b):
    msg = _edge_mlp(h_E, W_msg0, b_msg0)
    zeros = jnp.zeros((_N, _H), jnp.float32)
    partials = _scatter(msg, edge_idx[0], zeros)
    return _final(h_V, partials[0], partials[1], W_d0, b_d0, W_out, b_out,
                  ln1_w, ln1_b, ln2_w, ln2_b)


# speedup vs baseline: 3.1383x; 1.5354x over previous
import functools

import jax
import jax.numpy as jnp
from jax import lax
from jax.experimental import pallas as pl
from jax.experimental.pallas import tpu as pltpu
from jax.experimental.pallas import tpu_sc as plsc

_N, _E, _H, _HN = 10000, 320000, 128, 144
_BE = 2560
_BN = 2000
_NC, _NS = 2, 16
_EPW = _E // (_NC * _NS)
_C = 80
_NCH = _EPW // _C
_RPS = 624
_RLAST = _N - 15 * _RPS


def _gelu(x):
    return x * 0.5 * (1.0 + lax.erf(x * 0.7071067811865476))


def _edge_mlp_body(het_ref, w_ref, b_ref, out_ref):
    x = lax.dot_general(het_ref[...], w_ref[...], (((0,), (0,)), ((), ())),
                        preferred_element_type=jnp.float32)
    out_ref[...] = _gelu(x + b_ref[...])


def _edge_mlp(h_E_T, W, b):
    return pl.pallas_call(
        _edge_mlp_body,
        grid=(_E // _BE,),
        in_specs=[
            pl.BlockSpec((_HN, _BE), lambda i: (0, i)),
            pl.BlockSpec((_HN, _H), lambda i: (0, 0)),
            pl.BlockSpec((1, _H), lambda i: (0, 0)),
        ],
        out_specs=pl.BlockSpec((_BE, _H), lambda i: (i, 0)),
        out_shape=jax.ShapeDtypeStruct((_E, _H), jnp.float32),
    )(h_E_T, W, b.reshape(1, _H))


def _scatter_body(msg_hbm, idx_hbm, zero_hbm, out_hbm, rows_v, idx_v, acc_sh):
    c = lax.axis_index("c")
    s = lax.axis_index("s")
    r0 = pl.multiple_of(s * _RPS, 8)

    @pl.when(s < _NS - 1)
    def _():
        pltpu.sync_copy(zero_hbm.at[pl.ds(r0, _RPS)],
                        acc_sh.at[pl.ds(r0, _RPS)])

    @pl.when(s == _NS - 1)
    def _():
        pltpu.sync_copy(zero_hbm.at[pl.ds(r0, _RLAST)],
                        acc_sh.at[pl.ds(r0, _RLAST)])

    plsc.subcore_barrier()
    base = (s * _NC + c) * _EPW

    def body(i, carry):
        off = base + i * _C
        pltpu.sync_copy(idx_hbm.at[pl.ds(off, _C)], idx_v)
        pltpu.sync_copy(msg_hbm.at[pl.ds(off, _C)], rows_v)
        pltpu.sync_copy(rows_v, acc_sh.at[idx_v], add=True)
        return carry

    lax.fori_loop(0, _NCH, body, 0)
    plsc.subcore_barrier()

    @pl.when(s < _NS - 1)
    def _():
        pltpu.sync_copy(acc_sh.at[pl.ds(r0, _RPS)],
                        out_hbm.at[c, pl.ds(r0, _RPS)])

    @pl.when(s == _NS - 1)
    def _():
        pltpu.sync_copy(acc_sh.at[pl.ds(r0, _RLAST)],
                        out_hbm.at[c, pl.ds(r0, _RLAST)])


def _scatter(msg, src_idx, zeros):
    f = pl.kernel(
        _scatter_body,
        out_type=jax.ShapeDtypeStruct((_NC, _N, _H), jnp.float32),
        mesh=plsc.VectorSubcoreMesh(core_axis_name="c", subcore_axis_name="s"),
        scratch_types=[
            pltpu.VMEM((_C, _H), jnp.float32),
            pltpu.VMEM((_C,), jnp.int32),
            pltpu.VMEM_SHARED((_N, _H), jnp.float32),
        ],
    )
    return f(msg, src_idx, zeros)


def _final_body(hv_ref, p0_ref, p1_ref, wd_ref, bd_ref, wo_ref, bo_ref,
                ln1w_ref, ln1b_ref, ln2w_ref, ln2b_ref, out_ref):
    x = hv_ref[...] + (p0_ref[...] + p1_ref[...]) * (1.0 / 30.0)
    mu = jnp.mean(x, axis=-1, keepdims=True)
    xc = x - mu
    var = jnp.mean(xc * xc, axis=-1, keepdims=True)
    xn = xc * lax.rsqrt(var + 1e-5) * ln1w_ref[...] + ln1b_ref[...]
    h = _gelu(jnp.dot(xn, wd_ref[...], preferred_element_type=jnp.float32)
              + bd_ref[...])
    y = xn + jnp.dot(h, wo_ref[...], preferred_element_type=jnp.float32) \
        + bo_ref[...]
    mu2 = jnp.mean(y, axis=-1, keepdims=True)
    yc = y - mu2
    var2 = jnp.mean(yc * yc, axis=-1, keepdims=True)
    out_ref[...] = yc * lax.rsqrt(var2 + 1e-5) * ln2w_ref[...] + ln2b_ref[...]


def _final(h_V, p0, p1, W_d0, b_d0, W_out, b_out, ln1_w, ln1_b, ln2_w, ln2_b):
    row = pl.BlockSpec((_BN, _H), lambda i: (i, 0))
    full = pl.BlockSpec((_H, _H), lambda i: (0, 0))
    vec = pl.BlockSpec((1, _H), lambda i: (0, 0))
    return pl.pallas_call(
        _final_body,
        grid=(_N // _BN,),
        in_specs=[row, row, row, full, vec, full, vec, vec, vec, vec, vec],
        out_specs=row,
        out_shape=jax.ShapeDtypeStruct((_N, _H), jnp.float32),
    )(h_V, p0, p1, W_d0, b_d0.reshape(1, _H), W_out, b_out.reshape(1, _H),
      ln1_w.reshape(1, _H), ln1_b.reshape(1, _H),
      ln2_w.reshape(1, _H), ln2_b.reshape(1, _H))


def kernel(h_V, h_E, edge_idx, W_msg0, b_msg0, W_d0, b_d0, W_out, b_out,
           ln1_w, ln1_b, ln2_w, ln2_b):
    msg = _edge_mlp(h_E.T, W_msg0, b_msg0)
    zeros = jnp.zeros((_N, _H), jnp.float32)
    partials = _scatter(msg, edge_idx[0], zeros)
    return _final(h_V, partials[0], partials[1], W_d0, b_d0, W_out, b_out,
                  ln1_w, ln1_b, ln2_w, ln2_b)


# speedup vs baseline: 4.4048x; 1.4036x over previous
import functools

import jax
import jax.numpy as jnp
from jax import lax
from jax.experimental import pallas as pl
from jax.experimental.pallas import tpu as pltpu
from jax.experimental.pallas import tpu_sc as plsc

_N, _E, _H, _HN = 10000, 320000, 128, 144
_BE = 2560
_BN = 2000
_NC, _NS = 2, 16
_EPW = _E // (_NC * _NS)
_C = 80
_NCH = _EPW // _C
_RPS = 624
_RLAST = _N - 15 * _RPS


def _gelu(x):
    return x * 0.5 * (1.0 + lax.erf(x * 0.7071067811865476))


def _edge_mlp_body(het_ref, w_ref, b_ref, out_ref):
    x = lax.dot_general(het_ref[...], w_ref[...], (((0,), (0,)), ((), ())),
                        preferred_element_type=jnp.float32)
    out_ref[...] = _gelu(x + b_ref[...])


def _edge_mlp(h_E_T, W, b):
    return pl.pallas_call(
        _edge_mlp_body,
        grid=(_E // _BE,),
        in_specs=[
            pl.BlockSpec((_HN, _BE), lambda i: (0, i)),
            pl.BlockSpec((_HN, _H), lambda i: (0, 0)),
            pl.BlockSpec((1, _H), lambda i: (0, 0)),
        ],
        out_specs=pl.BlockSpec((_BE, _H), lambda i: (i, 0)),
        out_shape=jax.ShapeDtypeStruct((_E, _H), jnp.float32),
    )(h_E_T, W, b.reshape(1, _H))


def _scatter_body(msg_hbm, idx_hbm, zero_hbm, out_hbm,
                  rows0, rows1, idx0, idx1, acc_sh,
                  rsem0, rsem1, isem0, isem1):
    c = lax.axis_index("c")
    s = lax.axis_index("s")
    r0 = pl.multiple_of(s * _RPS, 8)

    @pl.when(s < _NS - 1)
    def _():
        pltpu.sync_copy(zero_hbm.at[pl.ds(r0, _RPS)],
                        acc_sh.at[pl.ds(r0, _RPS)])

    @pl.when(s == _NS - 1)
    def _():
        pltpu.sync_copy(zero_hbm.at[pl.ds(r0, _RLAST)],
                        acc_sh.at[pl.ds(r0, _RLAST)])

    plsc.subcore_barrier()
    base = (s * _NC + c) * _EPW
    bufs = ((rows0, idx0, rsem0, isem0), (rows1, idx1, rsem1, isem1))

    def start_load(k, b):
        rows_b, idx_b, rsem, isem = bufs[b]
        off = pl.multiple_of(base + k * _C, 8)
        pltpu.async_copy(msg_hbm.at[pl.ds(off, _C)], rows_b, rsem)
        pltpu.async_copy(idx_hbm.at[pl.ds(off, _C)], idx_b, isem)

    def consume(b):
        rows_b, idx_b, rsem, isem = bufs[b]
        pltpu.make_async_copy(msg_hbm.at[pl.ds(0, _C)], rows_b, rsem).wait()
        pltpu.make_async_copy(idx_hbm.at[pl.ds(0, _C)], idx_b, isem).wait()
        pltpu.sync_copy(rows_b, acc_sh.at[idx_b], add=True)

    start_load(0, 0)
    start_load(1, 1)

    def body(j, carry):
        consume(0)

        @pl.when(2 * j + 2 < _NCH)
        def _():
            start_load(2 * j + 2, 0)

        consume(1)

        @pl.when(2 * j + 3 < _NCH)
        def _():
            start_load(2 * j + 3, 1)

        return carry

    lax.fori_loop(0, (_NCH - 1) // 2, body, 0)
    consume(0)
    plsc.subcore_barrier()

    @pl.when(s < _NS - 1)
    def _():
        pltpu.sync_copy(acc_sh.at[pl.ds(r0, _RPS)],
                        out_hbm.at[c, pl.ds(r0, _RPS)])

    @pl.when(s == _NS - 1)
    def _():
        pltpu.sync_copy(acc_sh.at[pl.ds(r0, _RLAST)],
                        out_hbm.at[c, pl.ds(r0, _RLAST)])


def _scatter(msg, src_idx, zeros):
    f = pl.kernel(
        _scatter_body,
        out_type=jax.ShapeDtypeStruct((_NC, _N, _H), jnp.float32),
        mesh=plsc.VectorSubcoreMesh(core_axis_name="c", subcore_axis_name="s"),
        scratch_types=[
            pltpu.VMEM((_C, _H), jnp.float32),
            pltpu.VMEM((_C, _H), jnp.float32),
            pltpu.VMEM((_C,), jnp.int32),
            pltpu.VMEM((_C,), jnp.int32),
            pltpu.VMEM_SHARED((_N, _H), jnp.float32),
            pltpu.SemaphoreType.DMA,
            pltpu.SemaphoreType.DMA,
            pltpu.SemaphoreType.DMA,
            pltpu.SemaphoreType.DMA,
        ],
    )
    return f(msg, src_idx, zeros)


def _final_body(hv_ref, p0_ref, p1_ref, wd_ref, bd_ref, wo_ref, bo_ref,
                ln1w_ref, ln1b_ref, ln2w_ref, ln2b_ref, out_ref):
    x = hv_ref[...] + (p0_ref[...] + p1_ref[...]) * (1.0 / 30.0)
    mu = jnp.mean(x, axis=-1, keepdims=True)
    xc = x - mu
    var = jnp.mean(xc * xc, axis=-1, keepdims=True)
    xn = xc * lax.rsqrt(var + 1e-5) * ln1w_ref[...] + ln1b_ref[...]
    h = _gelu(jnp.dot(xn, wd_ref[...], preferred_element_type=jnp.float32)
              + bd_ref[...])
    y = xn + jnp.dot(h, wo_ref[...], preferred_element_type=jnp.float32) \
        + bo_ref[...]
    mu2 = jnp.mean(y, axis=-1, keepdims=True)
    yc = y - mu2
    var2 = jnp.mean(yc * yc, axis=-1, keepdims=True)
    out_ref[...] = yc * lax.rsqrt(var2 + 1e-5) * ln2w_ref[...] + ln2b_ref[...]


def _final(h_V, p0, p1, W_d0, b_d0, W_out, b_out, ln1_w, ln1_b, ln2_w, ln2_b):
    row = pl.BlockSpec((_BN, _H), lambda i: (i, 0))
    full = pl.BlockSpec((_H, _H), lambda i: (0, 0))
    vec = pl.BlockSpec((1, _H), lambda i: (0, 0))
    return pl.pallas_call(
        _final_body,
        grid=(_N // _BN,),
        in_specs=[row, row, row, full, vec, full, vec, vec, vec, vec, vec],
        out_specs=row,
        out_shape=jax.ShapeDtypeStruct((_N, _H), jnp.float32),
    )(h_V, p0, p1, W_d0, b_d0.reshape(1, _H), W_out, b_out.reshape(1, _H),
      ln1_w.reshape(1, _H), ln1_b.reshape(1, _H),
      ln2_w.reshape(1, _H), ln2_b.reshape(1, _H))


def kernel(h_V, h_E, edge_idx, W_msg0, b_msg0, W_d0, b_d0, W_out, b_out,
           ln1_w, ln1_b, ln2_w, ln2_b):
    msg = _edge_mlp(h_E.T, W_msg0, b_msg0)
    zeros = jnp.zeros((_N, _H), jnp.float32)
    partials = _scatter(msg, edge_idx[0], zeros)
    return _final(h_V, partials[0], partials[1], W_d0, b_d0, W_out, b_out,
                  ln1_w, ln1_b, ln2_w, ln2_b)


# speedup vs baseline: 4.4812x; 1.0173x over previous
import functools

import jax
import jax.numpy as jnp
from jax import lax
from jax.experimental import pallas as pl
from jax.experimental.pallas import tpu as pltpu
from jax.experimental.pallas import tpu_sc as plsc

_N, _E, _H, _HN = 10000, 320000, 128, 144
_BE = 2560
_BN = 2000
_NC, _NS = 2, 16
_EPW = _E // (_NC * _NS)
_C = 80
_NCH = _EPW // _C
_RPS = 624
_RLAST = _N - 15 * _RPS
_ZROWS = _RPS // 3


def _gelu(x):
    return x * 0.5 * (1.0 + lax.erf(x * 0.7071067811865476))


def _edge_mlp_body(het_ref, w_ref, b_ref, out_ref):
    x = lax.dot_general(het_ref[...], w_ref[...], (((0,), (0,)), ((), ())),
                        preferred_element_type=jnp.float32)
    out_ref[...] = _gelu(x + b_ref[...])


def _edge_mlp(h_E_T, W, b):
    return pl.pallas_call(
        _edge_mlp_body,
        grid=(_E // _BE,),
        in_specs=[
            pl.BlockSpec((_HN, _BE), lambda i: (0, i)),
            pl.BlockSpec((_HN, _H), lambda i: (0, 0)),
            pl.BlockSpec((1, _H), lambda i: (0, 0)),
        ],
        out_specs=pl.BlockSpec((_BE, _H), lambda i: (i, 0)),
        out_shape=jax.ShapeDtypeStruct((_E, _H), jnp.float32),
    )(h_E_T, W, b.reshape(1, _H))


def _scatter_body(msg_hbm, idx_hbm, out_hbm,
                  rows0, rows1, idx0, idx1, zbuf, acc_sh,
                  rsem0, rsem1, isem0, isem1, zsem):
    c = lax.axis_index("c")
    s = lax.axis_index("s")
    r0 = pl.multiple_of(s * _RPS, 8)
    base = (s * _NC + c) * _EPW
    bufs = ((rows0, idx0, rsem0, isem0), (rows1, idx1, rsem1, isem1))

    def start_load(k, b):
        rows_b, idx_b, rsem, isem = bufs[b]
        off = pl.multiple_of(base + k * _C, 8)
        pltpu.async_copy(msg_hbm.at[pl.ds(off, _C)], rows_b, rsem)
        pltpu.async_copy(idx_hbm.at[pl.ds(off, _C)], idx_b, isem)

    def consume(b):
        rows_b, idx_b, rsem, isem = bufs[b]
        pltpu.make_async_copy(msg_hbm.at[pl.ds(0, _C)], rows_b, rsem).wait()
        pltpu.make_async_copy(idx_hbm.at[pl.ds(0, _C)], idx_b, isem).wait()
        pltpu.sync_copy(rows_b, acc_sh.at[idx_b], add=True)

    start_load(0, 0)
    start_load(1, 1)

    def zrow(r, carry):
        for q in range(8):
            zbuf[r, pl.ds(q * 16, 16)] = jnp.zeros((16,), jnp.float32)
        return carry

    lax.fori_loop(0, _ZROWS, zrow, 0)
    for t in range(3):
        pltpu.async_copy(
            zbuf, acc_sh.at[pl.ds(pl.multiple_of(r0 + t * _ZROWS, 8),
                                  _ZROWS)], zsem)

    @pl.when(s == _NS - 1)
    def _():
        pltpu.async_copy(zbuf.at[pl.ds(0, 16)],
                         acc_sh.at[pl.ds(_N - 16, 16)], zsem)

    for t in range(3):
        pltpu.make_async_copy(zbuf, acc_sh.at[pl.ds(0, _ZROWS)], zsem).wait()

    @pl.when(s == _NS - 1)
    def _():
        pltpu.make_async_copy(zbuf.at[pl.ds(0, 16)],
                              acc_sh.at[pl.ds(0, 16)], zsem).wait()

    plsc.subcore_barrier()

    def body(j, carry):
        consume(0)

        @pl.when(2 * j + 2 < _NCH)
        def _():
            start_load(2 * j + 2, 0)

        consume(1)

        @pl.when(2 * j + 3 < _NCH)
        def _():
            start_load(2 * j + 3, 1)

        return carry

    lax.fori_loop(0, (_NCH - 1) // 2, body, 0)
    consume(0)
    plsc.subcore_barrier()

    @pl.when(s < _NS - 1)
    def _():
        pltpu.sync_copy(acc_sh.at[pl.ds(r0, _RPS)],
                        out_hbm.at[c, pl.ds(r0, _RPS)])

    @pl.when(s == _NS - 1)
    def _():
        pltpu.sync_copy(acc_sh.at[pl.ds(r0, _RLAST)],
                        out_hbm.at[c, pl.ds(r0, _RLAST)])


def _scatter(msg, src_idx):
    f = pl.kernel(
        _scatter_body,
        out_type=jax.ShapeDtypeStruct((_NC, _N, _H), jnp.float32),
        mesh=plsc.VectorSubcoreMesh(core_axis_name="c", subcore_axis_name="s"),
        scratch_types=[
            pltpu.VMEM((_C, _H), jnp.float32),
            pltpu.VMEM((_C, _H), jnp.float32),
            pltpu.VMEM((_C,), jnp.int32),
            pltpu.VMEM((_C,), jnp.int32),
            pltpu.VMEM((_ZROWS, _H), jnp.float32),
            pltpu.VMEM_SHARED((_N, _H), jnp.float32),
            pltpu.SemaphoreType.DMA,
            pltpu.SemaphoreType.DMA,
            pltpu.SemaphoreType.DMA,
            pltpu.SemaphoreType.DMA,
            pltpu.SemaphoreType.DMA,
        ],
    )
    return f(msg, src_idx)


def _final_body(hv_ref, p0_ref, p1_ref, wd_ref, bd_ref, wo_ref, bo_ref,
                ln1w_ref, ln1b_ref, ln2w_ref, ln2b_ref, out_ref):
    x = hv_ref[...] + (p0_ref[...] + p1_ref[...]) * (1.0 / 30.0)
    mu = jnp.mean(x, axis=-1, keepdims=True)
    xc = x - mu
    var = jnp.mean(xc * xc, axis=-1, keepdims=True)
    xn = xc * lax.rsqrt(var + 1e-5) * ln1w_ref[...] + ln1b_ref[...]
    h = _gelu(jnp.dot(xn, wd_ref[...], preferred_element_type=jnp.float32)
              + bd_ref[...])
    y = xn + jnp.dot(h, wo_ref[...], preferred_element_type=jnp.float32) \
        + bo_ref[...]
    mu2 = jnp.mean(y, axis=-1, keepdims=True)
    yc = y - mu2
    var2 = jnp.mean(yc * yc, axis=-1, keepdims=True)
    out_ref[...] = yc * lax.rsqrt(var2 + 1e-5) * ln2w_ref[...] + ln2b_ref[...]


def _final(h_V, p0, p1, W_d0, b_d0, W_out, b_out, ln1_w, ln1_b, ln2_w, ln2_b):
    row = pl.BlockSpec((_BN, _H), lambda i: (i, 0))
    full = pl.BlockSpec((_H, _H), lambda i: (0, 0))
    vec = pl.BlockSpec((1, _H), lambda i: (0, 0))
    return pl.pallas_call(
        _final_body,
        grid=(_N // _BN,),
        in_specs=[row, row, row, full, vec, full, vec, vec, vec, vec, vec],
        out_specs=row,
        out_shape=jax.ShapeDtypeStruct((_N, _H), jnp.float32),
    )(h_V, p0, p1, W_d0, b_d0.reshape(1, _H), W_out, b_out.reshape(1, _H),
      ln1_w.reshape(1, _H), ln1_b.reshape(1, _H),
      ln2_w.reshape(1, _H), ln2_b.reshape(1, _H))


def kernel(h_V, h_E, edge_idx, W_msg0, b_msg0, W_d0, b_d0, W_out, b_out,
           ln1_w, ln1_b, ln2_w, ln2_b):
    msg = _edge_mlp(h_E.T, W_msg0, b_msg0)
    partials = _scatter(msg, edge_idx[0])
    return _final(h_V, partials[0], partials[1], W_d0, b_d0, W_out, b_out,
                  ln1_w, ln1_b, ln2_w, ln2_b)


# speedup vs baseline: 4.6721x; 1.0426x over previous
import functools

import jax
import jax.numpy as jnp
from jax import lax
from jax.experimental import pallas as pl
from jax.experimental.pallas import tpu as pltpu
from jax.experimental.pallas import tpu_sc as plsc

_N, _E, _H, _HN = 10000, 320000, 128, 144
_BE = 2560
_BN = 2000
_NC, _NS = 2, 16
_EPW = _E // (_NC * _NS)
_C = 80
_NCH = _EPW // _C
_RPS = 624
_RLAST = _N - 15 * _RPS
_ZROWS = _RPS // 3


def _gelu(x):
    return x * 0.5 * (1.0 + lax.erf(x * 0.7071067811865476))


def _edge_mlp_body(het_ref, w_ref, b_ref, out_ref):
    x = lax.dot_general(het_ref[...], w_ref[...], (((0,), (0,)), ((), ())),
                        preferred_element_type=jnp.float32)
    out_ref[...] = _gelu(x + b_ref[...])


def _edge_mlp(h_E_T, W, b):
    return pl.pallas_call(
        _edge_mlp_body,
        grid=(_E // _BE,),
        in_specs=[
            pl.BlockSpec((_HN, _BE), lambda i: (0, i)),
            pl.BlockSpec((_HN, _H), lambda i: (0, 0)),
            pl.BlockSpec((1, _H), lambda i: (0, 0)),
        ],
        out_specs=pl.BlockSpec((_BE, _H), lambda i: (i, 0)),
        out_shape=jax.ShapeDtypeStruct((_E, _H), jnp.float32),
    )(h_E_T, W, b.reshape(1, _H))


def _scatter_body(msg_hbm, idx_hbm, out_hbm,
                  rows0, rows1, rows2, rows3, idx0, idx1, idx2, idx3, acc_sh,
                  rsem0, rsem1, rsem2, rsem3, isem0, isem1, isem2, isem3,
                  ssem0, ssem1, ssem2, ssem3, zsem):
    c = lax.axis_index("c")
    s = lax.axis_index("s")
    r0 = pl.multiple_of(s * _RPS, 8)
    base = (s * _NC + c) * _EPW
    rows = (rows0, rows1, rows2, rows3)
    idxs = (idx0, idx1, idx2, idx3)
    rsems = (rsem0, rsem1, rsem2, rsem3)
    isems = (isem0, isem1, isem2, isem3)
    ssems = (ssem0, ssem1, ssem2, ssem3)

    def start_load(k, b):
        off = pl.multiple_of(base + k * _C, 8)
        pltpu.async_copy(msg_hbm.at[pl.ds(off, _C)], rows[b], rsems[b])
        pltpu.async_copy(idx_hbm.at[pl.ds(off, _C)], idxs[b], isems[b])

    def wait_scatter(b):
        pltpu.make_async_copy(rows[b], acc_sh.at[idxs[b]], ssems[b]).wait()

    def step(k, b, first=False, load=True):
        if not first:
            wait_scatter((b + 2) % 4)
        if load:
            if isinstance(k, int):
                start_load(k + 2, (b + 2) % 4)
            else:
                @pl.when(k + 2 < _NCH)
                def _():
                    start_load(k + 2, (b + 2) % 4)
        pltpu.make_async_copy(msg_hbm.at[pl.ds(0, _C)], rows[b],
                              rsems[b]).wait()
        pltpu.make_async_copy(idx_hbm.at[pl.ds(0, _C)], idxs[b],
                              isems[b]).wait()
        pltpu.async_copy(rows[b], acc_sh.at[idxs[b]], ssems[b], add=True)

    start_load(0, 0)
    start_load(1, 1)

    def zrow(r, carry):
        for q in range(8):
            rows3[r, pl.ds(q * 16, 16)] = jnp.zeros((16,), jnp.float32)
        return carry

    lax.fori_loop(0, _C, zrow, 0)
    for t in range(7):
        pltpu.async_copy(
            rows3, acc_sh.at[pl.ds(pl.multiple_of(r0 + t * _C, 8), _C)],
            zsem)

    @pl.when(s < _NS - 1)
    def _():
        pltpu.async_copy(rows3.at[pl.ds(0, _RPS - 7 * _C)],
                         acc_sh.at[pl.ds(pl.multiple_of(r0 + 7 * _C, 8),
                                         _RPS - 7 * _C)], zsem)

    @pl.when(s == _NS - 1)
    def _():
        pltpu.async_copy(rows3,
                         acc_sh.at[pl.ds(pl.multiple_of(r0 + 7 * _C, 8),
                                         _C)], zsem)

    for t in range(7):
        pltpu.make_async_copy(rows3, acc_sh.at[pl.ds(0, _C)], zsem).wait()

    @pl.when(s < _NS - 1)
    def _():
        pltpu.make_async_copy(rows3.at[pl.ds(0, _RPS - 7 * _C)],
                              acc_sh.at[pl.ds(0, _RPS - 7 * _C)],
                              zsem).wait()

    @pl.when(s == _NS - 1)
    def _():
        pltpu.make_async_copy(rows3, acc_sh.at[pl.ds(0, _C)], zsem).wait()

    plsc.subcore_barrier()

    step(0, 0, first=True)
    step(1, 1, first=True)
    step(2, 2)
    step(3, 3)

    def body(j, carry):
        k = 4 * j
        for r in range(4):
            step(k + r, r)
        return carry

    lax.fori_loop(1, 31, body, 0)
    step(124, 0, load=False)
    wait_scatter(3)
    wait_scatter(0)
    plsc.subcore_barrier()

    @pl.when(s < _NS - 1)
    def _():
        pltpu.sync_copy(acc_sh.at[pl.ds(r0, _RPS)],
                        out_hbm.at[c, pl.ds(r0, _RPS)])

    @pl.when(s == _NS - 1)
    def _():
        pltpu.sync_copy(acc_sh.at[pl.ds(r0, _RLAST)],
                        out_hbm.at[c, pl.ds(r0, _RLAST)])


def _scatter(msg, src_idx):
    f = pl.kernel(
        _scatter_body,
        out_type=jax.ShapeDtypeStruct((_NC, _N, _H), jnp.float32),
        mesh=plsc.VectorSubcoreMesh(core_axis_name="c", subcore_axis_name="s"),
        scratch_types=(
            [pltpu.VMEM((_C, _H), jnp.float32) for _ in range(4)]
            + [pltpu.VMEM((_C,), jnp.int32) for _ in range(4)]
            + [pltpu.VMEM_SHARED((_N, _H), jnp.float32)]
            + [pltpu.SemaphoreType.DMA] * 13
        ),
    )
    return f(msg, src_idx)


def _final_body(hv_ref, p0_ref, p1_ref, wd_ref, bd_ref, wo_ref, bo_ref,
                ln1w_ref, ln1b_ref, ln2w_ref, ln2b_ref, out_ref):
    x = hv_ref[...] + (p0_ref[...] + p1_ref[...]) * (1.0 / 30.0)
    mu = jnp.mean(x, axis=-1, keepdims=True)
    xc = x - mu
    var = jnp.mean(xc * xc, axis=-1, keepdims=True)
    xn = xc * lax.rsqrt(var + 1e-5) * ln1w_ref[...] + ln1b_ref[...]
    h = _gelu(jnp.dot(xn, wd_ref[...], preferred_element_type=jnp.float32)
              + bd_ref[...])
    y = xn + jnp.dot(h, wo_ref[...], preferred_element_type=jnp.float32) \
        + bo_ref[...]
    mu2 = jnp.mean(y, axis=-1, keepdims=True)
    yc = y - mu2
    var2 = jnp.mean(yc * yc, axis=-1, keepdims=True)
    out_ref[...] = yc * lax.rsqrt(var2 + 1e-5) * ln2w_ref[...] + ln2b_ref[...]


def _final(h_V, p0, p1, W_d0, b_d0, W_out, b_out, ln1_w, ln1_b, ln2_w, ln2_b):
    row = pl.BlockSpec((_BN, _H), lambda i: (i, 0))
    full = pl.BlockSpec((_H, _H), lambda i: (0, 0))
    vec = pl.BlockSpec((1, _H), lambda i: (0, 0))
    return pl.pallas_call(
        _final_body,
        grid=(_N // _BN,),
        in_specs=[row, row, row, full, vec, full, vec, vec, vec, vec, vec],
        out_specs=row,
        out_shape=jax.ShapeDtypeStruct((_N, _H), jnp.float32),
    )(h_V, p0, p1, W_d0, b_d0.reshape(1, _H), W_out, b_out.reshape(1, _H),
      ln1_w.reshape(1, _H), ln1_b.reshape(1, _H),
      ln2_w.reshape(1, _H), ln2_b.reshape(1, _H))


def kernel(h_V, h_E, edge_idx, W_msg0, b_msg0, W_d0, b_d0, W_out, b_out,
           ln1_w, ln1_b, ln2_w, ln2_b):
    msg = _edge_mlp(h_E.T, W_msg0, b_msg0)
    partials = _scatter(msg, edge_idx[0])
    return _final(h_V, partials[0], partials[1], W_d0, b_d0, W_out, b_out,
                  ln1_w, ln1_b, ln2_w, ln2_b)
